# Initial kernel scaffold; baseline (speedup 1.0000x reference)
import jax
import jax.numpy as jnp
from jax.experimental import pallas as pl


def kernel(x, edge_index, edge_attr, u, batch, W1a, b1a, W1b, b1b, W2a, b2a, W2b, b2b):
    raise NotImplementedError("write your pallas kernel here")


# speedup vs baseline: 1.2746x; 1.2746x over previous
import jax
import jax.numpy as jnp
from jax import lax
from jax.experimental import pallas as pl
from jax.experimental.pallas import tpu as pltpu
from jax.experimental.pallas import tpu_sc as plsc

F32 = jnp.float32
_HIGHEST = lax.Precision.HIGHEST


def _dot(a, b):
    return jnp.dot(a, b, precision=_HIGHEST, preferred_element_type=F32)


def _node_premul(x, W1x, b1a):
    N, D = x.shape
    BN = 1000

    def body(x_ref, w_ref, b_ref, o_ref):
        o_ref[...] = _dot(x_ref[...], w_ref[...]) + b_ref[...]

    return pl.pallas_call(
        body,
        grid=(N // BN,),
        in_specs=[
            pl.BlockSpec((BN, D), lambda i: (i, 0)),
            pl.BlockSpec((D, D), lambda i: (0, 0)),
            pl.BlockSpec((1, D), lambda i: (0, 0)),
        ],
        out_specs=pl.BlockSpec((BN, D), lambda i: (i, 0)),
        out_shape=jax.ShapeDtypeStruct((N, D), F32),
    )(x, W1x, b1a)


def _sc_gather(xa, row, nc, ns, blk):
    N, D = xa.shape
    E = row.shape[0]
    nw = nc * ns
    epw = E // nw
    nblk = epw // blk

    mesh = plsc.VectorSubcoreMesh(core_axis_name="c", subcore_axis_name="s")

    @pl.kernel(
        out_type=jax.ShapeDtypeStruct((E, D), F32),
        mesh=mesh,
        scratch_types=[
            pltpu.VMEM((blk,), jnp.int32),
            pltpu.VMEM((blk, D), F32),
            pltpu.SemaphoreType.DMA,
        ],
    )
    def k(xa_hbm, row_hbm, out_hbm, idx_v, g_v, sem):
        c = lax.axis_index("c")
        s = lax.axis_index("s")
        base0 = (c * ns + s) * epw

        @pl.loop(0, nblk)
        def _(i):
            base = base0 + i * blk
            pltpu.sync_copy(row_hbm.at[pl.ds(base, blk)], idx_v)
            pltpu.async_copy(xa_hbm.at[idx_v], g_v, sem).wait()
            pltpu.sync_copy(g_v, out_hbm.at[pl.ds(base, blk)])

    return k(xa, row)


def _edge_mlp(g, ea, W1e):
    E, D = g.shape
    DE = ea.shape[1]
    BE = 4000

    def body(g_ref, ea_ref, w_ref, o_ref):
        o_ref[...] = jnp.maximum(g_ref[...] + _dot(ea_ref[...], w_ref[...]), 0.0)

    return pl.pallas_call(
        body,
        grid=(E // BE,),
        in_specs=[
            pl.BlockSpec((BE, D), lambda i: (i, 0)),
            pl.BlockSpec((BE, DE), lambda i: (i, 0)),
            pl.BlockSpec((DE, D), lambda i: (0, 0)),
        ],
        out_specs=pl.BlockSpec((BE, D), lambda i: (i, 0)),
        out_shape=jax.ShapeDtypeStruct((E, D), F32),
    )(g, ea, W1e)


def _sc_scatter(h, col, N, nc, ns, blk):
    E, D = h.shape
    NP = 2
    ZR = 40
    H = -(-N // (nc * NP * ZR)) * ZR
    HP = H + 8
    NPAD = nc * NP * H
    eps = E // ns
    nblk = eps // blk
    nzblk = H // ZR
    per_sub = -(-nzblk // ns)
    VC = 16

    zeros128 = jnp.zeros((ZR, D), F32)
    ones128 = jnp.ones((blk, D), F32)

    mesh = plsc.VectorSubcoreMesh(core_axis_name="c", subcore_axis_name="s")

    def one_pass(p):
        @pl.kernel(
            out_type=(
                jax.ShapeDtypeStruct((nc * H, D), F32),
                jax.ShapeDtypeStruct((nc * H, D), F32),
            ),
            mesh=mesh,
            scratch_types=[
                pltpu.VMEM((blk,), jnp.int32),
                pltpu.VMEM((blk, D), F32),
                pltpu.VMEM_SHARED((HP, D), F32),
                pltpu.SemaphoreType.DMA,
            ],
        )
        def k(h_hbm, col_hbm, z128_hbm, ones_hbm,
              sum_hbm, cnt_hbm, cidx_v, h_v, acc_s, sem):
            c = lax.axis_index("c")
            s = lax.axis_index("s")
            node0 = c * (NP * H) + p * H
            base0 = s * eps

            def zero_acc():
                @pl.loop(0, per_sub)
                def _(j):
                    b = s + j * ns

                    @pl.when(b < nzblk)
                    def _():
                        pltpu.sync_copy(z128_hbm, acc_s.at[pl.ds(b * ZR, ZR)])

            def rebase(i):
                base = base0 + i * blk
                pltpu.sync_copy(col_hbm.at[pl.ds(base, blk)], cidx_v)

                @pl.loop(0, blk, step=VC)
                def _(v):
                    idx = cidx_v[pl.ds(v, VC)] - node0
                    ok = (idx >= 0) & (idx < H)
                    cidx_v[pl.ds(v, VC)] = jnp.where(ok, idx, H)

            def dump(dst_hbm):
                @pl.loop(0, per_sub)
                def _(j):
                    b = s + j * ns

                    @pl.when(b < nzblk)
                    def _():
                        pltpu.sync_copy(acc_s.at[pl.ds(b * ZR, ZR)],
                                        dst_hbm.at[pl.ds(c * H + b * ZR, ZR)])

            zero_acc()
            plsc.subcore_barrier()

            @pl.loop(0, nblk)
            def _(i):
                rebase(i)
                base = base0 + i * blk
                pltpu.async_copy(h_hbm.at[pl.ds(base, blk)], h_v, sem).wait()
                pltpu.sync_copy(h_v, acc_s.at[cidx_v], add=True)

            plsc.subcore_barrier()
            dump(sum_hbm)
            plsc.subcore_barrier()

            zero_acc()
            pltpu.sync_copy(ones_hbm, h_v)
            plsc.subcore_barrier()

            @pl.loop(0, nblk)
            def _(i):
                rebase(i)
                pltpu.sync_copy(h_v, acc_s.at[cidx_v], add=True)

            plsc.subcore_barrier()
            dump(cnt_hbm)

        return k(h, col, zeros128, ones128)

    parts = [one_pass(p) for p in range(NP)]
    sums = (jnp.stack([q[0] for q in parts])
            .reshape(NP, nc, H, D).transpose(1, 0, 2, 3).reshape(NPAD, D))
    cnts = (jnp.stack([q[1] for q in parts])
            .reshape(NP, nc, H, D).transpose(1, 0, 2, 3).reshape(NPAD, D))
    return sums[:N], cnts[:N]


def _node_mlp2(x, sums, cnts, batch2d, u, W1b, b1b, W2x, W2m, W2u, b2a, W2b, b2b):
    N, D = x.shape
    NG, DU = u.shape
    CW = cnts.shape[1]
    BN = 1000

    def body(x_ref, s_ref, c_ref, bt_ref, u_ref, w1b_ref, b1b_ref,
             w2x_ref, w2m_ref, w2u_ref, b2a_ref, w2b_ref, b2b_ref, o_ref):
        s = s_ref[...]
        cnt = c_ref[:, :1]
        summed = _dot(s, w1b_ref[...]) + cnt * b1b_ref[...]
        mean = summed / jnp.maximum(cnt, 1.0)
        oh = (bt_ref[...] == lax.broadcasted_iota(jnp.int32, (1, NG), 1)).astype(F32)
        ug = _dot(oh, _dot(u_ref[...], w2u_ref[...]))
        h2 = jnp.maximum(
            _dot(x_ref[...], w2x_ref[...]) + _dot(mean, w2m_ref[...])
            + ug + b2a_ref[...], 0.0)
        o_ref[...] = _dot(h2, w2b_ref[...]) + b2b_ref[...]

    return pl.pallas_call(
        body,
        grid=(N // BN,),
        in_specs=[
            pl.BlockSpec((BN, D), lambda i: (i, 0)),
            pl.BlockSpec((BN, D), lambda i: (i, 0)),
            pl.BlockSpec((BN, CW), lambda i: (i, 0)),
            pl.BlockSpec((BN, 1), lambda i: (i, 0)),
            pl.BlockSpec((NG, DU), lambda i: (0, 0)),
            pl.BlockSpec((D, D), lambda i: (0, 0)),
            pl.BlockSpec((1, D), lambda i: (0, 0)),
            pl.BlockSpec((D, D), lambda i: (0, 0)),
            pl.BlockSpec((D, D), lambda i: (0, 0)),
            pl.BlockSpec((DU, D), lambda i: (0, 0)),
            pl.BlockSpec((1, D), lambda i: (0, 0)),
            pl.BlockSpec((D, D), lambda i: (0, 0)),
            pl.BlockSpec((1, D), lambda i: (0, 0)),
        ],
        out_specs=pl.BlockSpec((BN, D), lambda i: (i, 0)),
        out_shape=jax.ShapeDtypeStruct((N, D), F32),
    )(x, sums, cnts, batch2d, u, W1b, b1b, W2x, W2m, W2u, b2a, W2b, b2b)


def kernel(x, edge_index, edge_attr, u, batch, W1a, b1a, W1b, b1b, W2a, b2a, W2b, b2b):
    N, D = x.shape
    E = edge_index.shape[1]

    row = edge_index[0].astype(jnp.int32)
    col = edge_index[1].astype(jnp.int32)
    batch2d = batch.astype(jnp.int32).reshape(N, 1)
    b1a2 = b1a.reshape(1, -1)
    b1b2 = b1b.reshape(1, -1)
    b2a2 = b2a.reshape(1, -1)
    b2b2 = b2b.reshape(1, -1)
    W1x, W1e = W1a[:D], W1a[D:]
    W2x, W2m, W2u = W2a[:D], W2a[D : 2 * D], W2a[2 * D :]

    nc, ns = 2, 16
    blk = 200

    xa = _node_premul(x, W1x, b1a2)
    g = _sc_gather(xa, row, nc, ns, blk)
    h = _edge_mlp(g, edge_attr, W1e)
    sums, cnts = _sc_scatter(h, col, N, nc, ns, blk)
    return _node_mlp2(x, sums, cnts, batch2d, u, W1b, b1b2,
                      W2x, W2m, W2u, b2a2, W2b, b2b2)


# speedup vs baseline: 1.8001x; 1.4123x over previous
import dataclasses

import jax
import jax.numpy as jnp
from jax import lax
from jax.experimental import pallas as pl
from jax.experimental.pallas import tpu as pltpu
from jax.experimental.pallas import tpu_sc as plsc

F32 = jnp.float32
_HIGHEST = lax.Precision.HIGHEST


def _dot(a, b):
    return jnp.dot(a, b, precision=_HIGHEST, preferred_element_type=F32)


def _node_premul(x, W1x, b1a):
    N, D = x.shape
    BN = 1000

    def body(x_ref, w_ref, b_ref, o_ref):
        o_ref[...] = _dot(x_ref[...], w_ref[...]) + b_ref[...]

    return pl.pallas_call(
        body,
        grid=(N // BN,),
        in_specs=[
            pl.BlockSpec((BN, D), lambda i: (i, 0)),
            pl.BlockSpec((D, D), lambda i: (0, 0)),
            pl.BlockSpec((1, D), lambda i: (0, 0)),
        ],
        out_specs=pl.BlockSpec((BN, D), lambda i: (i, 0)),
        out_shape=jax.ShapeDtypeStruct((N, D), F32),
    )(x, W1x, b1a)


def _sc_gather(xa, row, nc, ns, blk):
    N, D = xa.shape
    E = row.shape[0]
    nw = nc * ns
    epw = E // nw
    nblk = epw // blk

    mesh = plsc.VectorSubcoreMesh(core_axis_name="c", subcore_axis_name="s")

    @pl.kernel(
        out_type=jax.ShapeDtypeStruct((E, D), F32),
        mesh=mesh,
        scratch_types=[
            pltpu.VMEM((blk,), jnp.int32),
            pltpu.VMEM((blk, D), F32),
            pltpu.SemaphoreType.DMA,
        ],
    )
    def k(xa_hbm, row_hbm, out_hbm, idx_v, g_v, sem):
        c = lax.axis_index("c")
        s = lax.axis_index("s")
        base0 = (c * ns + s) * epw

        @pl.loop(0, nblk)
        def _(i):
            base = base0 + i * blk
            pltpu.sync_copy(row_hbm.at[pl.ds(base, blk)], idx_v)
            pltpu.async_copy(xa_hbm.at[idx_v], g_v, sem).wait()
            pltpu.sync_copy(g_v, out_hbm.at[pl.ds(base, blk)])

    return k(xa, row)


def _edge_mlp(g, ea, W1e):
    E, D = g.shape
    DE = ea.shape[1]
    BE = 4000

    def body(g_ref, ea_ref, w_ref, o_ref):
        o_ref[...] = jnp.maximum(g_ref[...] + _dot(ea_ref[...], w_ref[...]), 0.0)

    return pl.pallas_call(
        body,
        grid=(E // BE,),
        in_specs=[
            pl.BlockSpec((BE, D), lambda i: (i, 0)),
            pl.BlockSpec((BE, DE), lambda i: (i, 0)),
            pl.BlockSpec((DE, D), lambda i: (0, 0)),
        ],
        out_specs=pl.BlockSpec((BE, D), lambda i: (i, 0)),
        out_shape=jax.ShapeDtypeStruct((E, D), F32),
    )(g, ea, W1e)


def _sc_scatter(h, col, N, nc, ns, blk):
    E, D = h.shape
    NP = 2
    ZR = 40
    H = -(-N // (nc * NP * ZR)) * ZR
    HP = H + 8
    HB = -(-(H + 1) // 128) * 128
    NPAD = nc * NP * H
    eps = E // ns
    nblk = eps // blk
    nzblk = H // ZR
    per_sub = -(-nzblk // ns)
    VC = 16

    zeros128 = jnp.zeros((ZR, D), F32)

    mesh = plsc.VectorSubcoreMesh(core_axis_name="c", subcore_axis_name="s")
    cp_params = pltpu.CompilerParams()
    if "needs_layout_passes" in pltpu.CompilerParams.__dataclass_fields__:
        cp_params = dataclasses.replace(cp_params, needs_layout_passes=False)

    def one_pass(p):
        @pl.kernel(
            out_type=(
                jax.ShapeDtypeStruct((nc * H, D), F32),
                jax.ShapeDtypeStruct((nc * H,), F32),
            ),
            mesh=mesh,
            compiler_params=cp_params,
            scratch_types=[
                pltpu.VMEM((blk,), jnp.int32),
                pltpu.VMEM((blk, D), F32),
                pltpu.VMEM((HB,), F32),
                pltpu.VMEM((HB,), F32),
                pltpu.VMEM_SHARED((HP, D), F32),
                pltpu.VMEM_SHARED((ns, HB), F32),
                pltpu.SemaphoreType.DMA,
            ],
        )
        def k(h_hbm, col_hbm, z128_hbm,
              sum_hbm, cnt_hbm, cidx_v, h_v, hist_v, tmp_v, acc_s,
              stage_s, sem):
            c = lax.axis_index("c")
            s = lax.axis_index("s")
            node0 = c * (NP * H) + p * H
            base0 = s * eps
            ones_v = jnp.ones((VC,), F32)

            @pl.loop(0, per_sub)
            def _(j):
                b = s + j * ns

                @pl.when(b < nzblk)
                def _():
                    pltpu.sync_copy(z128_hbm, acc_s.at[pl.ds(b * ZR, ZR)])

            @pl.loop(0, HB, step=VC)
            def _(v):
                hist_v[pl.ds(v, VC)] = jnp.zeros((VC,), F32)

            plsc.subcore_barrier()

            @pl.loop(0, nblk)
            def _(i):
                base = base0 + i * blk
                pltpu.sync_copy(col_hbm.at[pl.ds(base, blk)], cidx_v)
                cp = pltpu.async_copy(h_hbm.at[pl.ds(base, blk)], h_v, sem)

                @pl.loop(0, blk, step=VC)
                def _(v):
                    idx = cidx_v[pl.ds(v, VC)] - node0
                    ok = (idx >= 0) & (idx < H)
                    idx = jnp.where(ok, idx, H)
                    cidx_v[pl.ds(v, VC)] = idx
                    plsc.addupdate_scatter(hist_v, [idx], ones_v)

                cp.wait()
                pltpu.sync_copy(h_v, acc_s.at[cidx_v], add=True)

            plsc.subcore_barrier()

            @pl.loop(0, per_sub)
            def _(j):
                b = s + j * ns

                @pl.when(b < nzblk)
                def _():
                    pltpu.sync_copy(acc_s.at[pl.ds(b * ZR, ZR)],
                                    sum_hbm.at[pl.ds(c * H + b * ZR, ZR)])

            pltpu.sync_copy(hist_v, stage_s.at[s])
            plsc.subcore_barrier()

            @pl.when(s == 0)
            def _():
                @pl.loop(1, ns)
                def _(r):
                    pltpu.sync_copy(stage_s.at[r], tmp_v)

                    @pl.loop(0, HB, step=VC)
                    def _(v):
                        hist_v[pl.ds(v, VC)] = (hist_v[pl.ds(v, VC)]
                                                + tmp_v[pl.ds(v, VC)])

                pltpu.sync_copy(hist_v.at[pl.ds(0, H)],
                                cnt_hbm.at[pl.ds(c * H, H)])

        return k(h, col, zeros128)

    parts = [one_pass(p) for p in range(NP)]
    sums = (jnp.stack([q[0] for q in parts])
            .reshape(NP, nc, H, D).transpose(1, 0, 2, 3).reshape(NPAD, D))
    cnts = (jnp.stack([q[1] for q in parts])
            .reshape(NP, nc, H).transpose(1, 0, 2).reshape(NPAD,))
    return sums[:N], cnts[:N].reshape(N, 1)


def _node_mlp2(x, sums, cnts, batch2d, u, W1b, b1b, W2x, W2m, W2u, b2a, W2b, b2b):
    N, D = x.shape
    NG, DU = u.shape
    CW = cnts.shape[1]
    BN = 1000

    def body(x_ref, s_ref, c_ref, bt_ref, u_ref, w1b_ref, b1b_ref,
             w2x_ref, w2m_ref, w2u_ref, b2a_ref, w2b_ref, b2b_ref, o_ref):
        s = s_ref[...]
        cnt = c_ref[:, :1]
        summed = _dot(s, w1b_ref[...]) + cnt * b1b_ref[...]
        mean = summed / jnp.maximum(cnt, 1.0)
        oh = (bt_ref[...] == lax.broadcasted_iota(jnp.int32, (1, NG), 1)).astype(F32)
        ug = _dot(oh, _dot(u_ref[...], w2u_ref[...]))
        h2 = jnp.maximum(
            _dot(x_ref[...], w2x_ref[...]) + _dot(mean, w2m_ref[...])
            + ug + b2a_ref[...], 0.0)
        o_ref[...] = _dot(h2, w2b_ref[...]) + b2b_ref[...]

    return pl.pallas_call(
        body,
        grid=(N // BN,),
        in_specs=[
            pl.BlockSpec((BN, D), lambda i: (i, 0)),
            pl.BlockSpec((BN, D), lambda i: (i, 0)),
            pl.BlockSpec((BN, CW), lambda i: (i, 0)),
            pl.BlockSpec((BN, 1), lambda i: (i, 0)),
            pl.BlockSpec((NG, DU), lambda i: (0, 0)),
            pl.BlockSpec((D, D), lambda i: (0, 0)),
            pl.BlockSpec((1, D), lambda i: (0, 0)),
            pl.BlockSpec((D, D), lambda i: (0, 0)),
            pl.BlockSpec((D, D), lambda i: (0, 0)),
            pl.BlockSpec((DU, D), lambda i: (0, 0)),
            pl.BlockSpec((1, D), lambda i: (0, 0)),
            pl.BlockSpec((D, D), lambda i: (0, 0)),
            pl.BlockSpec((1, D), lambda i: (0, 0)),
        ],
        out_specs=pl.BlockSpec((BN, D), lambda i: (i, 0)),
        out_shape=jax.ShapeDtypeStruct((N, D), F32),
    )(x, sums, cnts, batch2d, u, W1b, b1b, W2x, W2m, W2u, b2a, W2b, b2b)


def kernel(x, edge_index, edge_attr, u, batch, W1a, b1a, W1b, b1b, W2a, b2a, W2b, b2b):
    N, D = x.shape
    E = edge_index.shape[1]

    row = edge_index[0].astype(jnp.int32)
    col = edge_index[1].astype(jnp.int32)
    batch2d = batch.astype(jnp.int32).reshape(N, 1)
    b1a2 = b1a.reshape(1, -1)
    b1b2 = b1b.reshape(1, -1)
    b2a2 = b2a.reshape(1, -1)
    b2b2 = b2b.reshape(1, -1)
    W1x, W1e = W1a[:D], W1a[D:]
    W2x, W2m, W2u = W2a[:D], W2a[D : 2 * D], W2a[2 * D :]

    nc, ns = 2, 16

    xa = _node_premul(x, W1x, b1a2)
    g = _sc_gather(xa, row, nc, ns, 200)
    h = _edge_mlp(g, edge_attr, W1e)
    sums, cnts = _sc_scatter(h, col, N, nc, ns, 160)
    return _node_mlp2(x, sums, cnts, batch2d, u, W1b, b1b2,
                      W2x, W2m, W2u, b2a2, W2b, b2b2)


# speedup vs baseline: 1.9442x; 1.0800x over previous
import dataclasses

import jax
import jax.numpy as jnp
from jax import lax
from jax.experimental import pallas as pl
from jax.experimental.pallas import tpu as pltpu
from jax.experimental.pallas import tpu_sc as plsc

F32 = jnp.float32
_HIGHEST = lax.Precision.HIGHEST


def _dot(a, b):
    return jnp.dot(a, b, precision=_HIGHEST, preferred_element_type=F32)


def _node_premul(x, W1x, b1a):
    N, D = x.shape
    BN = 1000

    def body(x_ref, w_ref, b_ref, o_ref):
        o_ref[...] = _dot(x_ref[...], w_ref[...]) + b_ref[...]

    return pl.pallas_call(
        body,
        grid=(N // BN,),
        in_specs=[
            pl.BlockSpec((BN, D), lambda i: (i, 0)),
            pl.BlockSpec((D, D), lambda i: (0, 0)),
            pl.BlockSpec((1, D), lambda i: (0, 0)),
        ],
        out_specs=pl.BlockSpec((BN, D), lambda i: (i, 0)),
        out_shape=jax.ShapeDtypeStruct((N, D), F32),
    )(x, W1x, b1a)


def _sc_gather(xa, row, nc, ns, blk):
    N, D = xa.shape
    E = row.shape[0]
    nw = nc * ns
    epw = E // nw
    nblk = epw // blk

    mesh = plsc.VectorSubcoreMesh(core_axis_name="c", subcore_axis_name="s")

    @pl.kernel(
        out_type=jax.ShapeDtypeStruct((E, D), F32),
        mesh=mesh,
        scratch_types=[
            pltpu.VMEM((blk,), jnp.int32),
            pltpu.VMEM((blk, D), F32),
            pltpu.SemaphoreType.DMA,
        ],
    )
    def k(xa_hbm, row_hbm, out_hbm, idx_v, g_v, sem):
        c = lax.axis_index("c")
        s = lax.axis_index("s")
        base0 = (c * ns + s) * epw

        @pl.loop(0, nblk)
        def _(i):
            base = base0 + i * blk
            pltpu.sync_copy(row_hbm.at[pl.ds(base, blk)], idx_v)
            pltpu.async_copy(xa_hbm.at[idx_v], g_v, sem).wait()
            pltpu.sync_copy(g_v, out_hbm.at[pl.ds(base, blk)])

    return k(xa, row)


def _edge_mlp(g, ea, W1e):
    E, D = g.shape
    DE = ea.shape[1]
    BE = 4000

    def body(g_ref, ea_ref, w_ref, o_ref):
        o_ref[...] = jnp.maximum(g_ref[...] + _dot(ea_ref[...], w_ref[...]), 0.0)

    return pl.pallas_call(
        body,
        grid=(E // BE,),
        in_specs=[
            pl.BlockSpec((BE, D), lambda i: (i, 0)),
            pl.BlockSpec((BE, DE), lambda i: (i, 0)),
            pl.BlockSpec((DE, D), lambda i: (0, 0)),
        ],
        out_specs=pl.BlockSpec((BE, D), lambda i: (i, 0)),
        out_shape=jax.ShapeDtypeStruct((E, D), F32),
    )(g, ea, W1e)


def _sc_scatter(h, col, N, nc, ns, blk):
    E, D = h.shape
    NP = 2
    ZR = 40
    H = -(-N // (nc * NP * ZR)) * ZR
    HP = H + 8
    HB = -(-(H + 1) // 128) * 128
    NPAD = nc * NP * H
    eps = E // ns
    nblk = eps // blk
    assert nblk % 2 == 1 and nblk * blk == eps
    nzblk = H // ZR
    per_sub = -(-nzblk // ns)
    VC = 16

    zeros128 = jnp.zeros((ZR, D), F32)

    mesh = plsc.VectorSubcoreMesh(core_axis_name="c", subcore_axis_name="s")
    cp_params = pltpu.CompilerParams()
    if "needs_layout_passes" in pltpu.CompilerParams.__dataclass_fields__:
        cp_params = dataclasses.replace(cp_params, needs_layout_passes=False)

    def one_pass(p):
        @pl.kernel(
            out_type=(
                jax.ShapeDtypeStruct((nc * H, D), F32),
                jax.ShapeDtypeStruct((nc * H,), F32),
            ),
            mesh=mesh,
            compiler_params=cp_params,
            scratch_types=[
                pltpu.VMEM((blk,), jnp.int32),
                pltpu.VMEM((blk,), jnp.int32),
                pltpu.VMEM((blk, D), F32),
                pltpu.VMEM((blk, D), F32),
                pltpu.VMEM((HB,), F32),
                pltpu.VMEM((HB,), F32),
                pltpu.VMEM_SHARED((HP, D), F32),
                pltpu.VMEM_SHARED((ns, HB), F32),
                pltpu.SemaphoreType.DMA,
                pltpu.SemaphoreType.DMA,
            ],
        )
        def k(h_hbm, col_hbm, z128_hbm,
              sum_hbm, cnt_hbm, cidx0_v, cidx1_v, h0_v, h1_v, hist_v,
              tmp_v, acc_s, stage_s, sem_h0, sem_h1):
            c = lax.axis_index("c")
            s = lax.axis_index("s")
            node0 = c * (NP * H) + p * H
            base0 = s * eps
            ones_v = jnp.ones((VC,), F32)

            @pl.loop(0, per_sub)
            def _(j):
                b = s + j * ns

                @pl.when(b < nzblk)
                def _():
                    pltpu.sync_copy(z128_hbm, acc_s.at[pl.ds(b * ZR, ZR)])

            @pl.loop(0, HB, step=VC)
            def _(v):
                hist_v[pl.ds(v, VC)] = jnp.zeros((VC,), F32)

            plsc.subcore_barrier()

            cidx = (cidx0_v, cidx1_v)
            hv = (h0_v, h1_v)
            sem_h = (sem_h0, sem_h1)

            def issue(i, b):
                base = base0 + i * blk
                pltpu.sync_copy(col_hbm.at[pl.ds(base, blk)], cidx[b])
                pltpu.async_copy(h_hbm.at[pl.ds(base, blk)], hv[b], sem_h[b])

            def rebase_hist(b):
                @pl.loop(0, blk, step=VC)
                def _(v):
                    idx = cidx[b][pl.ds(v, VC)] - node0
                    ok = (idx >= 0) & (idx < H)
                    idx = jnp.where(ok, idx, H)
                    cidx[b][pl.ds(v, VC)] = idx
                    plsc.addupdate_scatter(hist_v, [idx], ones_v)

            def drain_h(b):
                pltpu.make_async_copy(h_hbm.at[pl.ds(0, blk)], hv[b],
                                      sem_h[b]).wait()

            issue(0, 0)

            @pl.loop(0, nblk - 1, step=2)
            def _(i0):
                for b in (0, 1):
                    i = i0 + b
                    rebase_hist(b)

                    @pl.when(i + 1 < nblk)
                    def _():
                        issue(i + 1, 1 - b)

                    drain_h(b)
                    pltpu.sync_copy(hv[b], acc_s.at[cidx[b]], add=True)

            rebase_hist((nblk - 1) % 2)
            drain_h((nblk - 1) % 2)
            pltpu.sync_copy(hv[(nblk - 1) % 2],
                            acc_s.at[cidx[(nblk - 1) % 2]], add=True)

            plsc.subcore_barrier()

            @pl.loop(0, per_sub)
            def _(j):
                b = s + j * ns

                @pl.when(b < nzblk)
                def _():
                    pltpu.sync_copy(acc_s.at[pl.ds(b * ZR, ZR)],
                                    sum_hbm.at[pl.ds(c * H + b * ZR, ZR)])

            pltpu.sync_copy(hist_v, stage_s.at[s])
            plsc.subcore_barrier()

            @pl.when(s == 0)
            def _():
                @pl.loop(1, ns)
                def _(r):
                    pltpu.sync_copy(stage_s.at[r], tmp_v)

                    @pl.loop(0, HB, step=VC)
                    def _(v):
                        hist_v[pl.ds(v, VC)] = (hist_v[pl.ds(v, VC)]
                                                + tmp_v[pl.ds(v, VC)])

                pltpu.sync_copy(hist_v.at[pl.ds(0, H)],
                                cnt_hbm.at[pl.ds(c * H, H)])

        return k(h, col, zeros128)

    parts = [one_pass(p) for p in range(NP)]
    sums = (jnp.stack([q[0] for q in parts])
            .reshape(NP, nc, H, D).transpose(1, 0, 2, 3).reshape(NPAD, D))
    cnts = (jnp.stack([q[1] for q in parts])
            .reshape(NP, nc, H).transpose(1, 0, 2).reshape(NPAD,))
    return sums[:N], cnts[:N].reshape(N, 1)


def _node_mlp2(x, sums, cnts, batch2d, u, W1b, b1b, W2x, W2m, W2u, b2a, W2b, b2b):
    N, D = x.shape
    NG, DU = u.shape
    CW = cnts.shape[1]
    BN = 1000

    def body(x_ref, s_ref, c_ref, bt_ref, u_ref, w1b_ref, b1b_ref,
             w2x_ref, w2m_ref, w2u_ref, b2a_ref, w2b_ref, b2b_ref, o_ref):
        s = s_ref[...]
        cnt = c_ref[:, :1]
        summed = _dot(s, w1b_ref[...]) + cnt * b1b_ref[...]
        mean = summed / jnp.maximum(cnt, 1.0)
        oh = (bt_ref[...] == lax.broadcasted_iota(jnp.int32, (1, NG), 1)).astype(F32)
        ug = _dot(oh, _dot(u_ref[...], w2u_ref[...]))
        h2 = jnp.maximum(
            _dot(x_ref[...], w2x_ref[...]) + _dot(mean, w2m_ref[...])
            + ug + b2a_ref[...], 0.0)
        o_ref[...] = _dot(h2, w2b_ref[...]) + b2b_ref[...]

    return pl.pallas_call(
        body,
        grid=(N // BN,),
        in_specs=[
            pl.BlockSpec((BN, D), lambda i: (i, 0)),
            pl.BlockSpec((BN, D), lambda i: (i, 0)),
            pl.BlockSpec((BN, CW), lambda i: (i, 0)),
            pl.BlockSpec((BN, 1), lambda i: (i, 0)),
            pl.BlockSpec((NG, DU), lambda i: (0, 0)),
            pl.BlockSpec((D, D), lambda i: (0, 0)),
            pl.BlockSpec((1, D), lambda i: (0, 0)),
            pl.BlockSpec((D, D), lambda i: (0, 0)),
            pl.BlockSpec((D, D), lambda i: (0, 0)),
            pl.BlockSpec((DU, D), lambda i: (0, 0)),
            pl.BlockSpec((1, D), lambda i: (0, 0)),
            pl.BlockSpec((D, D), lambda i: (0, 0)),
            pl.BlockSpec((1, D), lambda i: (0, 0)),
        ],
        out_specs=pl.BlockSpec((BN, D), lambda i: (i, 0)),
        out_shape=jax.ShapeDtypeStruct((N, D), F32),
    )(x, sums, cnts, batch2d, u, W1b, b1b, W2x, W2m, W2u, b2a, W2b, b2b)


def kernel(x, edge_index, edge_attr, u, batch, W1a, b1a, W1b, b1b, W2a, b2a, W2b, b2b):
    N, D = x.shape
    E = edge_index.shape[1]

    row = edge_index[0].astype(jnp.int32)
    col = edge_index[1].astype(jnp.int32)
    batch2d = batch.astype(jnp.int32).reshape(N, 1)
    b1a2 = b1a.reshape(1, -1)
    b1b2 = b1b.reshape(1, -1)
    b2a2 = b2a.reshape(1, -1)
    b2b2 = b2b.reshape(1, -1)
    W1x, W1e = W1a[:D], W1a[D:]
    W2x, W2m, W2u = W2a[:D], W2a[D : 2 * D], W2a[2 * D :]

    nc, ns = 2, 16

    xa = _node_premul(x, W1x, b1a2)
    g = _sc_gather(xa, row, nc, ns, 200)
    h = _edge_mlp(g, edge_attr, W1e)
    sums, cnts = _sc_scatter(h, col, N, nc, ns, 160)
    return _node_mlp2(x, sums, cnts, batch2d, u, W1b, b1b2,
                      W2x, W2m, W2u, b2a2, W2b, b2b2)


# speedup vs baseline: 1.9816x; 1.0192x over previous
import dataclasses

import jax
import jax.numpy as jnp
from jax import lax
from jax.experimental import pallas as pl
from jax.experimental.pallas import tpu as pltpu
from jax.experimental.pallas import tpu_sc as plsc

F32 = jnp.float32
_HIGHEST = lax.Precision.HIGHEST


def _dot(a, b):
    return jnp.dot(a, b, precision=_HIGHEST, preferred_element_type=F32)


def _node_premul(x, W1x, b1a):
    N, D = x.shape
    BN = 1000

    def body(x_ref, w_ref, b_ref, o_ref):
        o_ref[...] = _dot(x_ref[...], w_ref[...]) + b_ref[...]

    return pl.pallas_call(
        body,
        grid=(N // BN,),
        in_specs=[
            pl.BlockSpec((BN, D), lambda i: (i, 0)),
            pl.BlockSpec((D, D), lambda i: (0, 0)),
            pl.BlockSpec((1, D), lambda i: (0, 0)),
        ],
        out_specs=pl.BlockSpec((BN, D), lambda i: (i, 0)),
        out_shape=jax.ShapeDtypeStruct((N, D), F32),
    )(x, W1x, b1a)


def _sc_gather(xa, row, nc, ns, blk):
    N, D = xa.shape
    E = row.shape[0]
    nw = nc * ns
    epw = E // nw
    nblk = epw // blk

    mesh = plsc.VectorSubcoreMesh(core_axis_name="c", subcore_axis_name="s")

    assert nblk % 2 == 0 and nblk * blk == epw

    @pl.kernel(
        out_type=jax.ShapeDtypeStruct((E, D), F32),
        mesh=mesh,
        scratch_types=[
            pltpu.VMEM((blk,), jnp.int32),
            pltpu.VMEM((blk,), jnp.int32),
            pltpu.VMEM((blk, D), F32),
            pltpu.VMEM((blk, D), F32),
            pltpu.SemaphoreType.DMA,
            pltpu.SemaphoreType.DMA,
        ],
    )
    def k(xa_hbm, row_hbm, out_hbm, idx0_v, idx1_v, g0_v, g1_v,
          sem0, sem1):
        c = lax.axis_index("c")
        s = lax.axis_index("s")
        base0 = (c * ns + s) * epw
        idx = (idx0_v, idx1_v)
        gv = (g0_v, g1_v)
        sem = (sem0, sem1)

        def issue(i, b):
            base = base0 + i * blk
            pltpu.sync_copy(row_hbm.at[pl.ds(base, blk)], idx[b])
            pltpu.async_copy(xa_hbm.at[idx[b]], gv[b], sem[b])

        issue(0, 0)

        @pl.loop(0, nblk, step=2)
        def _(i0):
            for b in (0, 1):
                i = i0 + b

                @pl.when(i + 1 < nblk)
                def _():
                    issue(i + 1, 1 - b)

                pltpu.make_async_copy(xa_hbm.at[pl.ds(0, blk)], gv[b],
                                      sem[b]).wait()
                base = base0 + i * blk
                pltpu.sync_copy(gv[b], out_hbm.at[pl.ds(base, blk)])

    return k(xa, row)


def _edge_mlp(g, ea, W1e):
    E, D = g.shape
    DE = ea.shape[1]
    BE = 4000

    def body(g_ref, ea_ref, w_ref, o_ref):
        o_ref[...] = jnp.maximum(g_ref[...] + _dot(ea_ref[...], w_ref[...]), 0.0)

    return pl.pallas_call(
        body,
        grid=(E // BE,),
        in_specs=[
            pl.BlockSpec((BE, D), lambda i: (i, 0)),
            pl.BlockSpec((BE, DE), lambda i: (i, 0)),
            pl.BlockSpec((DE, D), lambda i: (0, 0)),
        ],
        out_specs=pl.BlockSpec((BE, D), lambda i: (i, 0)),
        out_shape=jax.ShapeDtypeStruct((E, D), F32),
    )(g, ea, W1e)


def _sc_scatter(h, col, N, nc, ns, blk):
    E, D = h.shape
    NP = 2
    ZR = 40
    H = -(-N // (nc * NP * ZR)) * ZR
    HP = H + 8
    HB = -(-(H + 1) // 128) * 128
    NPAD = nc * NP * H
    eps = E // ns
    nblk = eps // blk
    assert nblk % 2 == 1 and nblk * blk == eps
    nzblk = H // ZR
    per_sub = -(-nzblk // ns)
    VC = 16

    zeros128 = jnp.zeros((ZR, D), F32)

    mesh = plsc.VectorSubcoreMesh(core_axis_name="c", subcore_axis_name="s")
    cp_params = pltpu.CompilerParams()
    if "needs_layout_passes" in pltpu.CompilerParams.__dataclass_fields__:
        cp_params = dataclasses.replace(cp_params, needs_layout_passes=False)

    def one_pass(p):
        @pl.kernel(
            out_type=(
                jax.ShapeDtypeStruct((nc * H, D), F32),
                jax.ShapeDtypeStruct((nc * H,), F32),
            ),
            mesh=mesh,
            compiler_params=cp_params,
            scratch_types=[
                pltpu.VMEM((blk,), jnp.int32),
                pltpu.VMEM((blk,), jnp.int32),
                pltpu.VMEM((blk, D), F32),
                pltpu.VMEM((blk, D), F32),
                pltpu.VMEM((HB,), F32),
                pltpu.VMEM((HB,), F32),
                pltpu.VMEM_SHARED((HP, D), F32),
                pltpu.VMEM_SHARED((ns, HB), F32),
                pltpu.SemaphoreType.DMA,
                pltpu.SemaphoreType.DMA,
            ],
        )
        def k(h_hbm, col_hbm, z128_hbm,
              sum_hbm, cnt_hbm, cidx0_v, cidx1_v, h0_v, h1_v, hist_v,
              tmp_v, acc_s, stage_s, sem_h0, sem_h1):
            c = lax.axis_index("c")
            s = lax.axis_index("s")
            node0 = c * (NP * H) + p * H
            base0 = s * eps
            ones_v = jnp.ones((VC,), F32)

            @pl.loop(0, per_sub)
            def _(j):
                b = s + j * ns

                @pl.when(b < nzblk)
                def _():
                    pltpu.sync_copy(z128_hbm, acc_s.at[pl.ds(b * ZR, ZR)])

            @pl.loop(0, HB, step=VC)
            def _(v):
                hist_v[pl.ds(v, VC)] = jnp.zeros((VC,), F32)

            plsc.subcore_barrier()

            cidx = (cidx0_v, cidx1_v)
            hv = (h0_v, h1_v)
            sem_h = (sem_h0, sem_h1)

            def issue(i, b):
                base = base0 + i * blk
                pltpu.sync_copy(col_hbm.at[pl.ds(base, blk)], cidx[b])
                pltpu.async_copy(h_hbm.at[pl.ds(base, blk)], hv[b], sem_h[b])

            def rebase_hist(b):
                @pl.loop(0, blk, step=VC)
                def _(v):
                    idx = cidx[b][pl.ds(v, VC)] - node0
                    ok = (idx >= 0) & (idx < H)
                    idx = jnp.where(ok, idx, H)
                    cidx[b][pl.ds(v, VC)] = idx
                    plsc.addupdate_scatter(hist_v, [idx], ones_v)

            def drain_h(b):
                pltpu.make_async_copy(h_hbm.at[pl.ds(0, blk)], hv[b],
                                      sem_h[b]).wait()

            issue(0, 0)

            @pl.loop(0, nblk - 1, step=2)
            def _(i0):
                for b in (0, 1):
                    i = i0 + b
                    rebase_hist(b)

                    @pl.when(i + 1 < nblk)
                    def _():
                        issue(i + 1, 1 - b)

                    drain_h(b)
                    pltpu.sync_copy(hv[b], acc_s.at[cidx[b]], add=True)

            rebase_hist((nblk - 1) % 2)
            drain_h((nblk - 1) % 2)
            pltpu.sync_copy(hv[(nblk - 1) % 2],
                            acc_s.at[cidx[(nblk - 1) % 2]], add=True)

            plsc.subcore_barrier()

            @pl.loop(0, per_sub)
            def _(j):
                b = s + j * ns

                @pl.when(b < nzblk)
                def _():
                    pltpu.sync_copy(acc_s.at[pl.ds(b * ZR, ZR)],
                                    sum_hbm.at[pl.ds(c * H + b * ZR, ZR)])

            pltpu.sync_copy(hist_v, stage_s.at[s])
            plsc.subcore_barrier()

            @pl.when(s == 0)
            def _():
                @pl.loop(1, ns)
                def _(r):
                    pltpu.sync_copy(stage_s.at[r], tmp_v)

                    @pl.loop(0, HB, step=VC)
                    def _(v):
                        hist_v[pl.ds(v, VC)] = (hist_v[pl.ds(v, VC)]
                                                + tmp_v[pl.ds(v, VC)])

                pltpu.sync_copy(hist_v.at[pl.ds(0, H)],
                                cnt_hbm.at[pl.ds(c * H, H)])

        return k(h, col, zeros128)

    parts = [one_pass(p) for p in range(NP)]
    sums = (jnp.stack([q[0] for q in parts])
            .reshape(NP, nc, H, D).transpose(1, 0, 2, 3).reshape(NPAD, D))
    cnts = (jnp.stack([q[1] for q in parts])
            .reshape(NP, nc, H).transpose(1, 0, 2).reshape(NPAD,))
    return sums[:N], cnts[:N].reshape(N, 1)


def _node_mlp2(x, sums, cnts, batch2d, u, W1b, b1b, W2x, W2m, W2u, b2a, W2b, b2b):
    N, D = x.shape
    NG, DU = u.shape
    CW = cnts.shape[1]
    BN = 1000

    def body(x_ref, s_ref, c_ref, bt_ref, u_ref, w1b_ref, b1b_ref,
             w2x_ref, w2m_ref, w2u_ref, b2a_ref, w2b_ref, b2b_ref, o_ref):
        s = s_ref[...]
        cnt = c_ref[:, :1]
        summed = _dot(s, w1b_ref[...]) + cnt * b1b_ref[...]
        mean = summed / jnp.maximum(cnt, 1.0)
        oh = (bt_ref[...] == lax.broadcasted_iota(jnp.int32, (1, NG), 1)).astype(F32)
        ug = _dot(oh, _dot(u_ref[...], w2u_ref[...]))
        h2 = jnp.maximum(
            _dot(x_ref[...], w2x_ref[...]) + _dot(mean, w2m_ref[...])
            + ug + b2a_ref[...], 0.0)
        o_ref[...] = _dot(h2, w2b_ref[...]) + b2b_ref[...]

    return pl.pallas_call(
        body,
        grid=(N // BN,),
        in_specs=[
            pl.BlockSpec((BN, D), lambda i: (i, 0)),
            pl.BlockSpec((BN, D), lambda i: (i, 0)),
            pl.BlockSpec((BN, CW), lambda i: (i, 0)),
            pl.BlockSpec((BN, 1), lambda i: (i, 0)),
            pl.BlockSpec((NG, DU), lambda i: (0, 0)),
            pl.BlockSpec((D, D), lambda i: (0, 0)),
            pl.BlockSpec((1, D), lambda i: (0, 0)),
            pl.BlockSpec((D, D), lambda i: (0, 0)),
            pl.BlockSpec((D, D), lambda i: (0, 0)),
            pl.BlockSpec((DU, D), lambda i: (0, 0)),
            pl.BlockSpec((1, D), lambda i: (0, 0)),
            pl.BlockSpec((D, D), lambda i: (0, 0)),
            pl.BlockSpec((1, D), lambda i: (0, 0)),
        ],
        out_specs=pl.BlockSpec((BN, D), lambda i: (i, 0)),
        out_shape=jax.ShapeDtypeStruct((N, D), F32),
    )(x, sums, cnts, batch2d, u, W1b, b1b, W2x, W2m, W2u, b2a, W2b, b2b)


def kernel(x, edge_index, edge_attr, u, batch, W1a, b1a, W1b, b1b, W2a, b2a, W2b, b2b):
    N, D = x.shape
    E = edge_index.shape[1]

    row = edge_index[0].astype(jnp.int32)
    col = edge_index[1].astype(jnp.int32)
    batch2d = batch.astype(jnp.int32).reshape(N, 1)
    b1a2 = b1a.reshape(1, -1)
    b1b2 = b1b.reshape(1, -1)
    b2a2 = b2a.reshape(1, -1)
    b2b2 = b2b.reshape(1, -1)
    W1x, W1e = W1a[:D], W1a[D:]
    W2x, W2m, W2u = W2a[:D], W2a[D : 2 * D], W2a[2 * D :]

    nc, ns = 2, 16

    xa = _node_premul(x, W1x, b1a2)
    g = _sc_gather(xa, row, nc, ns, 200)
    h = _edge_mlp(g, edge_attr, W1e)
    sums, cnts = _sc_scatter(h, col, N, nc, ns, 160)
    return _node_mlp2(x, sums, cnts, batch2d, u, W1b, b1b2,
                      W2x, W2m, W2u, b2a2, W2b, b2b2)
